# Initial kernel scaffold; baseline (speedup 1.0000x reference)
import jax
import jax.numpy as jnp
from jax.experimental import pallas as pl


def kernel(hidden_states, W_router, ln1_g, ln1_b, c_attn_w, c_attn_b, attn_proj_w, attn_proj_b, ln2_g, ln2_b, c_fc_w, c_fc_b, mlp_proj_w, mlp_proj_b):
    raise NotImplementedError("write your pallas kernel here")


# speedup vs baseline: 2.2143x; 2.2143x over previous
import functools

import jax
import jax.numpy as jnp
import numpy as np
from jax import lax
from jax.experimental import pallas as pl
from jax.experimental.pallas import tpu as pltpu
from jax.experimental.pallas import tpu_sc as plsc

S, D, H, CAP = 4096, 2048, 16, 2048
INNER = 4 * D
HD = D // H
EPS = 1e-5
SQ2PI = 0.7978845608028654
NEG_HUGE = float(np.finfo(np.float32).min)

_f32 = jnp.float32
_i32 = jnp.int32


def _ln(x, g, b):
    mu = jnp.mean(x, axis=-1, keepdims=True)
    var = jnp.mean(jnp.square(x - mu), axis=-1, keepdims=True)
    return (x - mu) / jnp.sqrt(var + EPS) * g + b


def _gelu_new(x):
    return 0.5 * x * (1.0 + jnp.tanh(SQ2PI * (x + 0.044715 * x * x * x)))


def _router_kernel(x_ref, w_ref, o_ref):
    o_ref[...] = lax.dot_general(
        x_ref[...], w_ref[...], (((1,), (1,)), ((), ())),
        preferred_element_type=_f32)


def _router(hs2, w):
    BM, BN = 512, 512
    return pl.pallas_call(
        _router_kernel,
        grid=(S // BM, D // BN),
        in_specs=[pl.BlockSpec((BM, D), lambda i, j: (i, 0)),
                  pl.BlockSpec((BN, D), lambda i, j: (j, 0))],
        out_specs=pl.BlockSpec((BM, BN), lambda i, j: (i, j)),
        out_shape=jax.ShapeDtypeStruct((S, D), _f32),
    )(hs2, w)


SORT_CS = 128


def _sort_kernel(r_ref, vals_ref, gidx_ref):
    C = r_ref.shape[1]
    col0 = pl.program_id(0) * C
    iota_s = lax.broadcasted_iota(_i32, (S, 1), 0)
    k_ = r_ref[...]
    i_ = lax.broadcasted_iota(_i32, (S, C), 0)
    kk = 2
    while kk <= S:
        jj = kk // 2
        while jj >= 1:
            up = (iota_s & jj) == 0
            xb = ((iota_s // kk) ^ (iota_s // jj)) & 1
            keep_prec = xb == 0
            pk = jnp.where(up, pltpu.roll(k_, S - jj, axis=0),
                           pltpu.roll(k_, jj, axis=0))
            pi = jnp.where(up, pltpu.roll(i_, S - jj, axis=0),
                           pltpu.roll(i_, jj, axis=0))
            prec = (k_ > pk) | ((k_ == pk) & (i_ < pi))
            lo_k = jnp.where(prec, k_, pk)
            hi_k = jnp.where(prec, pk, k_)
            lo_i = jnp.where(prec, i_, pi)
            hi_i = jnp.where(prec, pi, i_)
            k_ = jnp.where(keep_prec, lo_k, hi_k)
            i_ = jnp.where(keep_prec, lo_i, hi_i)
            jj //= 2
        kk *= 2
    lane = lax.broadcasted_iota(_i32, (CAP, C), 1)
    vals_ref[...] = k_[:CAP, :]
    gidx_ref[...] = i_[:CAP, :] * D + (col0 + lane)


def _sort_topk(r):
    return pl.pallas_call(
        _sort_kernel,
        grid=(D // SORT_CS,),
        in_specs=[pl.BlockSpec((S, SORT_CS), lambda c: (0, c))],
        out_specs=[pl.BlockSpec((CAP, SORT_CS), lambda c: (0, c)),
                   pl.BlockSpec((CAP, SORT_CS), lambda c: (0, c))],
        out_shape=[jax.ShapeDtypeStruct((CAP, D), _f32),
                   jax.ShapeDtypeStruct((CAP, D), _i32)],
    )(r)


GW = 32
G_ROWS = CAP * D // GW // 128
G_JB = 64
G_OUTER = G_ROWS // G_JB


def _gather_body(hs_f, gidx2, out2, idx_v, buf_v, sem):
    w = lax.axis_index("s") * 2 + lax.axis_index("c")
    row0 = w * G_ROWS

    def body(t, carry):
        r0 = row0 + t * G_JB
        pltpu.sync_copy(gidx2.at[pl.ds(r0, G_JB)], idx_v)
        handles = [pltpu.async_copy(hs_f.at[idx_v.at[j]], buf_v.at[j], sem)
                   for j in range(G_JB)]
        for h in handles:
            h.wait()
        pltpu.sync_copy(buf_v, out2.at[pl.ds(r0, G_JB)])
        return carry

    lax.fori_loop(0, G_OUTER, body, 0)


def _gather_sc(hs_f, gidx2):
    fn = pl.kernel(
        _gather_body,
        out_type=jax.ShapeDtypeStruct((CAP * D // 128, 128), _f32),
        mesh=plsc.VectorSubcoreMesh(core_axis_name="c", subcore_axis_name="s"),
        scratch_types=[pltpu.VMEM((G_JB, 128), _i32),
                       pltpu.VMEM((G_JB, 128), _f32),
                       pltpu.SemaphoreType.DMA],
    )
    return fn(hs_f, gidx2)


SW = 16
CP_ELEMS = S * D // SW
CP_CHUNK = 32768
CP_OUTER = CP_ELEMS // CP_CHUNK
SC_ROWS = CAP * D // SW // 128
SC_JB = 64
SC_OUTER = SC_ROWS // SC_JB


def _scatter_body(hs_f, hid2, gidx2, out_f, cbuf, idx_v, hbuf, sem):
    w = lax.axis_index("s")
    base = w * CP_ELEMS

    def cbody(t, carry):
        off = base + t * CP_CHUNK
        pltpu.sync_copy(hs_f.at[pl.ds(off, CP_CHUNK)], cbuf)
        pltpu.sync_copy(cbuf, out_f.at[pl.ds(off, CP_CHUNK)])
        return carry

    lax.fori_loop(0, CP_OUTER, cbody, 0)
    plsc.subcore_barrier()
    row0 = w * SC_ROWS

    def sbody(t, carry):
        r0 = row0 + t * SC_JB
        pltpu.sync_copy(gidx2.at[pl.ds(r0, SC_JB)], idx_v)
        pltpu.sync_copy(hid2.at[pl.ds(r0, SC_JB)], hbuf)
        handles = [pltpu.async_copy(hbuf.at[j], out_f.at[idx_v.at[j]], sem)
                   for j in range(SC_JB)]
        for h in handles:
            h.wait()
        return carry

    lax.fori_loop(0, SC_OUTER, sbody, 0)


def _scatter_sc(hs_f, hid2, gidx2):
    fn = pl.kernel(
        _scatter_body,
        out_type=jax.ShapeDtypeStruct((S * D,), _f32),
        mesh=plsc.VectorSubcoreMesh(core_axis_name="c", subcore_axis_name="s",
                                    num_cores=1),
        scratch_types=[pltpu.VMEM((CP_CHUNK,), _f32),
                       pltpu.VMEM((SC_JB, 128), _i32),
                       pltpu.VMEM((SC_JB, 128), _f32),
                       pltpu.SemaphoreType.DMA],
    )
    return fn(hs_f, hid2, gidx2)


def _lnmm_kernel(x_ref, g_ref, b_ref, w_ref, bias_ref, o_ref):
    x = _ln(x_ref[...], g_ref[...], b_ref[...])
    o_ref[...] = lax.dot_general(
        x, w_ref[...], (((1,), (0,)), ((), ())),
        preferred_element_type=_f32) + bias_ref[...]


def _lnmm(x, g, b, w, bias, n_out):
    BM, BN = 512, 512
    M = x.shape[0]
    return pl.pallas_call(
        _lnmm_kernel,
        grid=(M // BM, n_out // BN),
        in_specs=[pl.BlockSpec((BM, D), lambda i, j: (i, 0)),
                  pl.BlockSpec((1, D), lambda i, j: (0, 0)),
                  pl.BlockSpec((1, D), lambda i, j: (0, 0)),
                  pl.BlockSpec((D, BN), lambda i, j: (0, j)),
                  pl.BlockSpec((1, BN), lambda i, j: (0, j))],
        out_specs=pl.BlockSpec((BM, BN), lambda i, j: (i, j)),
        out_shape=jax.ShapeDtypeStruct((M, n_out), _f32),
    )(x, g, b, w, bias)


ATT_BQ = 512


def _attn_kernel(q_ref, k_ref, v_ref, o_ref):
    iq = pl.program_id(1)
    s = lax.dot_general(q_ref[...], k_ref[...], (((1,), (1,)), ((), ())),
                        preferred_element_type=_f32)
    s = s * (1.0 / float(np.sqrt(HD)))
    ri = iq * ATT_BQ + lax.broadcasted_iota(_i32, (ATT_BQ, CAP), 0)
    ci = lax.broadcasted_iota(_i32, (ATT_BQ, CAP), 1)
    s = jnp.where(ci <= ri, s, NEG_HUGE)
    m = jnp.max(s, axis=1, keepdims=True)
    e = jnp.exp(s - m)
    p = e / jnp.sum(e, axis=1, keepdims=True)
    o_ref[...] = lax.dot_general(p, v_ref[...], (((1,), (0,)), ((), ())),
                                 preferred_element_type=_f32)


def _attention(qkv):
    return pl.pallas_call(
        _attn_kernel,
        grid=(H, CAP // ATT_BQ),
        in_specs=[pl.BlockSpec((ATT_BQ, HD), lambda h, iq: (iq, h)),
                  pl.BlockSpec((CAP, HD), lambda h, iq: (0, H + h)),
                  pl.BlockSpec((CAP, HD), lambda h, iq: (0, 2 * H + h))],
        out_specs=pl.BlockSpec((ATT_BQ, HD), lambda h, iq: (iq, h)),
        out_shape=jax.ShapeDtypeStruct((CAP, D), _f32),
    )(qkv, qkv, qkv)


def _projres_kernel(x_ref, w_ref, bias_ref, res_ref, o_ref):
    o_ref[...] = lax.dot_general(
        x_ref[...], w_ref[...], (((1,), (0,)), ((), ())),
        preferred_element_type=_f32) + bias_ref[...] + res_ref[...]


def _projres(x, w, bias, res):
    BM, BN = 512, 512
    return pl.pallas_call(
        _projres_kernel,
        grid=(CAP // BM, D // BN),
        in_specs=[pl.BlockSpec((BM, D), lambda i, j: (i, 0)),
                  pl.BlockSpec((D, BN), lambda i, j: (0, j)),
                  pl.BlockSpec((1, BN), lambda i, j: (0, j)),
                  pl.BlockSpec((BM, BN), lambda i, j: (i, j))],
        out_specs=pl.BlockSpec((BM, BN), lambda i, j: (i, j)),
        out_shape=jax.ShapeDtypeStruct((CAP, D), _f32),
    )(x, w, bias, res)


def _lnfc_kernel(x_ref, g_ref, b_ref, w_ref, bias_ref, o_ref):
    x = _ln(x_ref[...], g_ref[...], b_ref[...])
    y = lax.dot_general(x, w_ref[...], (((1,), (0,)), ((), ())),
                        preferred_element_type=_f32) + bias_ref[...]
    o_ref[...] = _gelu_new(y)


def _lnfc(x, g, b, w, bias):
    BM, BN = 512, 512
    return pl.pallas_call(
        _lnfc_kernel,
        grid=(CAP // BM, INNER // BN),
        in_specs=[pl.BlockSpec((BM, D), lambda i, j: (i, 0)),
                  pl.BlockSpec((1, D), lambda i, j: (0, 0)),
                  pl.BlockSpec((1, D), lambda i, j: (0, 0)),
                  pl.BlockSpec((D, BN), lambda i, j: (0, j)),
                  pl.BlockSpec((1, BN), lambda i, j: (0, j))],
        out_specs=pl.BlockSpec((BM, BN), lambda i, j: (i, j)),
        out_shape=jax.ShapeDtypeStruct((CAP, INNER), _f32),
    )(x, g, b, w, bias)


def _mlpproj_kernel(ff_ref, w_ref, bias_ref, res_ref, vals_ref, o_ref):
    y = lax.dot_general(ff_ref[...], w_ref[...], (((1,), (0,)), ((), ())),
                        preferred_element_type=_f32) + bias_ref[...]
    o_ref[...] = res_ref[...] + y * vals_ref[...]


def _mlpproj(ff, w, bias, res, vals):
    BM, BN = 256, 256
    return pl.pallas_call(
        _mlpproj_kernel,
        grid=(CAP // BM, D // BN),
        in_specs=[pl.BlockSpec((BM, INNER), lambda i, j: (i, 0)),
                  pl.BlockSpec((INNER, BN), lambda i, j: (0, j)),
                  pl.BlockSpec((1, BN), lambda i, j: (0, j)),
                  pl.BlockSpec((BM, BN), lambda i, j: (i, j)),
                  pl.BlockSpec((BM, BN), lambda i, j: (i, j))],
        out_specs=pl.BlockSpec((BM, BN), lambda i, j: (i, j)),
        out_shape=jax.ShapeDtypeStruct((CAP, D), _f32),
    )(ff, w, bias, res, vals)


def kernel(hidden_states, W_router, ln1_g, ln1_b, c_attn_w, c_attn_b,
           attn_proj_w, attn_proj_b, ln2_g, ln2_b, c_fc_w, c_fc_b,
           mlp_proj_w, mlp_proj_b):
    hs2 = hidden_states.reshape(S, D)
    hs_f = hidden_states.reshape(S * D)

    r = _router(hs2, W_router)
    vals, gidx = _sort_topk(r)
    gidx2 = gidx.reshape(CAP * D // 128, 128)

    sel = _gather_sc(hs_f, gidx2).reshape(CAP, D)

    qkv = _lnmm(sel, ln1_g.reshape(1, D), ln1_b.reshape(1, D),
                c_attn_w, c_attn_b.reshape(1, 3 * D), 3 * D)
    ctx = _attention(qkv)
    hidden = _projres(ctx, attn_proj_w, attn_proj_b.reshape(1, D), sel)
    ff = _lnfc(hidden, ln2_g.reshape(1, D), ln2_b.reshape(1, D),
               c_fc_w, c_fc_b.reshape(1, INNER))
    hid = _mlpproj(ff, mlp_proj_w, mlp_proj_b.reshape(1, D), hidden, vals)

    out_f = _scatter_sc(hs_f, hid.reshape(CAP * D // 128, 128), gidx2)
    return out_f.reshape(1, S, D)


# speedup vs baseline: 2.8662x; 1.2944x over previous
import functools

import jax
import jax.numpy as jnp
import numpy as np
from jax import lax
from jax.experimental import pallas as pl
from jax.experimental.pallas import tpu as pltpu
from jax.experimental.pallas import tpu_sc as plsc

S, D, H, CAP = 4096, 2048, 16, 2048
INNER = 4 * D
HD = D // H
EPS = 1e-5
SQ2PI = 0.7978845608028654
NEG_HUGE = float(np.finfo(np.float32).min)

_f32 = jnp.float32
_i32 = jnp.int32


def _ln(x, g, b):
    mu = jnp.mean(x, axis=-1, keepdims=True)
    var = jnp.mean(jnp.square(x - mu), axis=-1, keepdims=True)
    return (x - mu) / jnp.sqrt(var + EPS) * g + b


def _gelu_new(x):
    return 0.5 * x * (1.0 + jnp.tanh(SQ2PI * (x + 0.044715 * x * x * x)))


def _router_kernel(x_ref, w_ref, o_ref):
    o_ref[...] = lax.dot_general(
        x_ref[...], w_ref[...], (((1,), (1,)), ((), ())),
        preferred_element_type=_f32)


def _router(hs2, w):
    BM, BN = 512, 512
    return pl.pallas_call(
        _router_kernel,
        grid=(S // BM, D // BN),
        in_specs=[pl.BlockSpec((BM, D), lambda i, j: (i, 0)),
                  pl.BlockSpec((BN, D), lambda i, j: (j, 0))],
        out_specs=pl.BlockSpec((BM, BN), lambda i, j: (i, j)),
        out_shape=jax.ShapeDtypeStruct((S, D), _f32),
    )(hs2, w)


SORT_CS = 128


def _sort_kernel(r_ref, vals_ref, gidx_ref, ifull_ref):
    C = r_ref.shape[1]
    col0 = pl.program_id(0) * C
    iota_s = lax.broadcasted_iota(_i32, (S, 1), 0)
    k_ = r_ref[...]
    i_ = lax.broadcasted_iota(_i32, (S, C), 0)
    kk = 2
    while kk <= S:
        jj = kk // 2
        while jj >= 1:
            up = (iota_s & jj) == 0
            xb = ((iota_s // kk) ^ (iota_s // jj)) & 1
            keep_prec = xb == 0
            pk = jnp.where(up, pltpu.roll(k_, S - jj, axis=0),
                           pltpu.roll(k_, jj, axis=0))
            pi = jnp.where(up, pltpu.roll(i_, S - jj, axis=0),
                           pltpu.roll(i_, jj, axis=0))
            prec = (k_ > pk) | ((k_ == pk) & (i_ < pi))
            lo_k = jnp.where(prec, k_, pk)
            hi_k = jnp.where(prec, pk, k_)
            lo_i = jnp.where(prec, i_, pi)
            hi_i = jnp.where(prec, pi, i_)
            k_ = jnp.where(keep_prec, lo_k, hi_k)
            i_ = jnp.where(keep_prec, lo_i, hi_i)
            jj //= 2
        kk *= 2
    lane = lax.broadcasted_iota(_i32, (CAP, C), 1)
    vals_ref[...] = k_[:CAP, :]
    gidx_ref[...] = i_[:CAP, :] * D + (col0 + lane)
    ifull_ref[...] = i_


def _invert_kernel(ifull_ref, g2_ref):
    C = ifull_ref.shape[1]
    col0 = pl.program_id(0) * C
    iota_s = lax.broadcasted_iota(_i32, (S, 1), 0)
    c2 = lax.broadcasted_iota(_i32, (S, C), 0)
    p_ = ifull_ref[...] * S + c2
    kk = 2
    while kk <= S:
        jj = kk // 2
        while jj >= 1:
            up = (iota_s & jj) == 0
            keep_lo = (((iota_s // kk) ^ (iota_s // jj)) & 1) == 0
            pp = jnp.where(up, pltpu.roll(p_, S - jj, axis=0),
                           pltpu.roll(p_, jj, axis=0))
            lo = jnp.minimum(p_, pp)
            hi = jnp.maximum(p_, pp)
            p_ = jnp.where(keep_lo, lo, hi)
            jj //= 2
        kk *= 2
    rank = p_ & (S - 1)
    lane_s = lax.broadcasted_iota(_i32, (S, C), 1) + col0
    n_flat = c2 * D + lane_s
    g2_ref[...] = jnp.where(rank < CAP, rank * D + lane_s, CAP * D + n_flat)


def _sort_topk(r):
    vals, gidx, ifull = pl.pallas_call(
        _sort_kernel,
        grid=(D // SORT_CS,),
        in_specs=[pl.BlockSpec((S, SORT_CS), lambda c: (0, c))],
        out_specs=[pl.BlockSpec((CAP, SORT_CS), lambda c: (0, c)),
                   pl.BlockSpec((CAP, SORT_CS), lambda c: (0, c)),
                   pl.BlockSpec((S, SORT_CS), lambda c: (0, c))],
        out_shape=[jax.ShapeDtypeStruct((CAP, D), _f32),
                   jax.ShapeDtypeStruct((CAP, D), _i32),
                   jax.ShapeDtypeStruct((S, D), _i32)],
    )(r)
    g2 = pl.pallas_call(
        _invert_kernel,
        grid=(D // SORT_CS,),
        in_specs=[pl.BlockSpec((S, SORT_CS), lambda c: (0, c))],
        out_specs=pl.BlockSpec((S, SORT_CS), lambda c: (0, c)),
        out_shape=jax.ShapeDtypeStruct((S, D), _i32),
    )(ifull)
    return vals, gidx, g2


GW = 32
G_JB = 64


def _make_gather_body(n_rows):
    rows_per_w = n_rows // GW
    n_outer = rows_per_w // G_JB

    def body(src_f, gidx2, out2, idx_v, buf_v, sem):
        w = lax.axis_index("s") * 2 + lax.axis_index("c")
        row0 = w * rows_per_w

        def inner(t, carry):
            r0 = row0 + t * G_JB
            pltpu.sync_copy(gidx2.at[pl.ds(r0, G_JB)], idx_v)
            handles = [pltpu.async_copy(src_f.at[idx_v.at[j]], buf_v.at[j], sem)
                       for j in range(G_JB)]
            for h in handles:
                h.wait()
            pltpu.sync_copy(buf_v, out2.at[pl.ds(r0, G_JB)])
            return carry

        lax.fori_loop(0, n_outer, inner, 0)

    return body


def _gather_sc(src_f, gidx2):
    n_rows = gidx2.shape[0]
    fn = pl.kernel(
        _make_gather_body(n_rows),
        out_type=jax.ShapeDtypeStruct((n_rows, 128), _f32),
        mesh=plsc.VectorSubcoreMesh(core_axis_name="c", subcore_axis_name="s"),
        scratch_types=[pltpu.VMEM((G_JB, 128), _i32),
                       pltpu.VMEM((G_JB, 128), _f32),
                       pltpu.SemaphoreType.DMA],
    )
    return fn(src_f, gidx2)


def _lnmm_kernel(x_ref, g_ref, b_ref, w_ref, bias_ref, o_ref):
    x = _ln(x_ref[...], g_ref[...], b_ref[...])
    o_ref[...] = lax.dot_general(
        x, w_ref[...], (((1,), (0,)), ((), ())),
        preferred_element_type=_f32) + bias_ref[...]


def _lnmm(x, g, b, w, bias, n_out):
    BM, BN = 512, 512
    M = x.shape[0]
    return pl.pallas_call(
        _lnmm_kernel,
        grid=(M // BM, n_out // BN),
        in_specs=[pl.BlockSpec((BM, D), lambda i, j: (i, 0)),
                  pl.BlockSpec((1, D), lambda i, j: (0, 0)),
                  pl.BlockSpec((1, D), lambda i, j: (0, 0)),
                  pl.BlockSpec((D, BN), lambda i, j: (0, j)),
                  pl.BlockSpec((1, BN), lambda i, j: (0, j))],
        out_specs=pl.BlockSpec((BM, BN), lambda i, j: (i, j)),
        out_shape=jax.ShapeDtypeStruct((M, n_out), _f32),
    )(x, g, b, w, bias)


ATT_BQ = 512


def _attn_kernel(q_ref, k_ref, v_ref, o_ref):
    iq = pl.program_id(1)
    s = lax.dot_general(q_ref[...], k_ref[...], (((1,), (1,)), ((), ())),
                        preferred_element_type=_f32)
    s = s * (1.0 / float(np.sqrt(HD)))
    ri = iq * ATT_BQ + lax.broadcasted_iota(_i32, (ATT_BQ, CAP), 0)
    ci = lax.broadcasted_iota(_i32, (ATT_BQ, CAP), 1)
    s = jnp.where(ci <= ri, s, NEG_HUGE)
    m = jnp.max(s, axis=1, keepdims=True)
    e = jnp.exp(s - m)
    p = e / jnp.sum(e, axis=1, keepdims=True)
    o_ref[...] = lax.dot_general(p, v_ref[...], (((1,), (0,)), ((), ())),
                                 preferred_element_type=_f32)


def _attention(qkv):
    return pl.pallas_call(
        _attn_kernel,
        grid=(H, CAP // ATT_BQ),
        in_specs=[pl.BlockSpec((ATT_BQ, HD), lambda h, iq: (iq, h)),
                  pl.BlockSpec((CAP, HD), lambda h, iq: (0, H + h)),
                  pl.BlockSpec((CAP, HD), lambda h, iq: (0, 2 * H + h))],
        out_specs=pl.BlockSpec((ATT_BQ, HD), lambda h, iq: (iq, h)),
        out_shape=jax.ShapeDtypeStruct((CAP, D), _f32),
    )(qkv, qkv, qkv)


def _projres_kernel(x_ref, w_ref, bias_ref, res_ref, o_ref):
    o_ref[...] = lax.dot_general(
        x_ref[...], w_ref[...], (((1,), (0,)), ((), ())),
        preferred_element_type=_f32) + bias_ref[...] + res_ref[...]


def _projres(x, w, bias, res):
    BM, BN = 512, 512
    return pl.pallas_call(
        _projres_kernel,
        grid=(CAP // BM, D // BN),
        in_specs=[pl.BlockSpec((BM, D), lambda i, j: (i, 0)),
                  pl.BlockSpec((D, BN), lambda i, j: (0, j)),
                  pl.BlockSpec((1, BN), lambda i, j: (0, j)),
                  pl.BlockSpec((BM, BN), lambda i, j: (i, j))],
        out_specs=pl.BlockSpec((BM, BN), lambda i, j: (i, j)),
        out_shape=jax.ShapeDtypeStruct((CAP, D), _f32),
    )(x, w, bias, res)


def _lnfc_kernel(x_ref, g_ref, b_ref, w_ref, bias_ref, o_ref):
    x = _ln(x_ref[...], g_ref[...], b_ref[...])
    y = lax.dot_general(x, w_ref[...], (((1,), (0,)), ((), ())),
                        preferred_element_type=_f32) + bias_ref[...]
    o_ref[...] = _gelu_new(y)


def _lnfc(x, g, b, w, bias):
    BM, BN = 512, 512
    return pl.pallas_call(
        _lnfc_kernel,
        grid=(CAP // BM, INNER // BN),
        in_specs=[pl.BlockSpec((BM, D), lambda i, j: (i, 0)),
                  pl.BlockSpec((1, D), lambda i, j: (0, 0)),
                  pl.BlockSpec((1, D), lambda i, j: (0, 0)),
                  pl.BlockSpec((D, BN), lambda i, j: (0, j)),
                  pl.BlockSpec((1, BN), lambda i, j: (0, j))],
        out_specs=pl.BlockSpec((BM, BN), lambda i, j: (i, j)),
        out_shape=jax.ShapeDtypeStruct((CAP, INNER), _f32),
    )(x, g, b, w, bias)


def _mlpproj_kernel(ff_ref, w_ref, bias_ref, res_ref, vals_ref, o_ref):
    y = lax.dot_general(ff_ref[...], w_ref[...], (((1,), (0,)), ((), ())),
                        preferred_element_type=_f32) + bias_ref[...]
    o_ref[...] = res_ref[...] + y * vals_ref[...]


def _mlpproj(ff, w, bias, res, vals):
    BM, BN = 256, 256
    return pl.pallas_call(
        _mlpproj_kernel,
        grid=(CAP // BM, D // BN),
        in_specs=[pl.BlockSpec((BM, INNER), lambda i, j: (i, 0)),
                  pl.BlockSpec((INNER, BN), lambda i, j: (0, j)),
                  pl.BlockSpec((1, BN), lambda i, j: (0, j)),
                  pl.BlockSpec((BM, BN), lambda i, j: (i, j)),
                  pl.BlockSpec((BM, BN), lambda i, j: (i, j))],
        out_specs=pl.BlockSpec((BM, BN), lambda i, j: (i, j)),
        out_shape=jax.ShapeDtypeStruct((CAP, D), _f32),
    )(ff, w, bias, res, vals)


def kernel(hidden_states, W_router, ln1_g, ln1_b, c_attn_w, c_attn_b,
           attn_proj_w, attn_proj_b, ln2_g, ln2_b, c_fc_w, c_fc_b,
           mlp_proj_w, mlp_proj_b):
    hs2 = hidden_states.reshape(S, D)
    hs_f = hidden_states.reshape(S * D)

    r = _router(hs2, W_router)
    vals, gidx, g2 = _sort_topk(r)
    gidx2 = gidx.reshape(CAP * D // 128, 128)

    sel = _gather_sc(hs_f, gidx2).reshape(CAP, D)

    qkv = _lnmm(sel, ln1_g.reshape(1, D), ln1_b.reshape(1, D),
                c_attn_w, c_attn_b.reshape(1, 3 * D), 3 * D)
    ctx = _attention(qkv)
    hidden = _projres(ctx, attn_proj_w, attn_proj_b.reshape(1, D), sel)
    ff = _lnfc(hidden, ln2_g.reshape(1, D), ln2_b.reshape(1, D),
               c_fc_w, c_fc_b.reshape(1, INNER))
    hid = _mlpproj(ff, mlp_proj_w, mlp_proj_b.reshape(1, D), hidden, vals)

    src_f = jnp.concatenate([hid.reshape(CAP * D), hs_f])
    out2 = _gather_sc(src_f, g2.reshape(S * D // 128, 128))
    return out2.reshape(1, S, D)


# speedup vs baseline: 3.8812x; 1.3541x over previous
import functools

import jax
import jax.numpy as jnp
import numpy as np
from jax import lax
from jax.experimental import pallas as pl
from jax.experimental.pallas import tpu as pltpu
from jax.experimental.pallas import tpu_sc as plsc

S, D, H, CAP = 4096, 2048, 16, 2048
INNER = 4 * D
HD = D // H
EPS = 1e-5
SQ2PI = 0.7978845608028654
NEG_HUGE = float(np.finfo(np.float32).min)

_f32 = jnp.float32
_i32 = jnp.int32


def _ln(x, g, b):
    mu = jnp.mean(x, axis=-1, keepdims=True)
    var = jnp.mean(jnp.square(x - mu), axis=-1, keepdims=True)
    return (x - mu) / jnp.sqrt(var + EPS) * g + b


def _gelu_new(x):
    return 0.5 * x * (1.0 + jnp.tanh(SQ2PI * (x + 0.044715 * x * x * x)))


def _router_kernel(x_ref, w_ref, o_ref):
    o_ref[...] = lax.dot_general(
        x_ref[...], w_ref[...], (((1,), (1,)), ((), ())),
        preferred_element_type=_f32)


def _router(hs2, w):
    BM, BN = 512, 512
    return pl.pallas_call(
        _router_kernel,
        grid=(S // BM, D // BN),
        in_specs=[pl.BlockSpec((BM, D), lambda i, j: (i, 0)),
                  pl.BlockSpec((BN, D), lambda i, j: (j, 0))],
        out_specs=pl.BlockSpec((BM, BN), lambda i, j: (i, j)),
        out_shape=jax.ShapeDtypeStruct((S, D), _f32),
    )(hs2, w)


SORT_CS = 128


def _sort_kernel(r_ref, vals_ref, gidx_ref, ifull_ref):
    C = r_ref.shape[1]
    col0 = pl.program_id(0) * C
    iota_s = lax.broadcasted_iota(_i32, (S, 1), 0)
    k_ = r_ref[...]
    i_ = lax.broadcasted_iota(_i32, (S, C), 0)
    kk = 2
    while kk <= S:
        jj = kk // 2
        while jj >= 1:
            up = (iota_s & jj) == 0
            xb = ((iota_s // kk) ^ (iota_s // jj)) & 1
            keep_prec = xb == 0
            pk = jnp.where(up, pltpu.roll(k_, S - jj, axis=0),
                           pltpu.roll(k_, jj, axis=0))
            pi = jnp.where(up, pltpu.roll(i_, S - jj, axis=0),
                           pltpu.roll(i_, jj, axis=0))
            prec = (k_ > pk) | ((k_ == pk) & (i_ < pi))
            lo_k = jnp.where(prec, k_, pk)
            hi_k = jnp.where(prec, pk, k_)
            lo_i = jnp.where(prec, i_, pi)
            hi_i = jnp.where(prec, pi, i_)
            k_ = jnp.where(keep_prec, lo_k, hi_k)
            i_ = jnp.where(keep_prec, lo_i, hi_i)
            jj //= 2
        kk *= 2
    lane = lax.broadcasted_iota(_i32, (CAP, C), 1)
    vals_ref[...] = k_[:CAP, :]
    gidx_ref[...] = i_[:CAP, :] * D + (col0 + lane)
    ifull_ref[...] = i_


def _sort_topk(r):
    return pl.pallas_call(
        _sort_kernel,
        grid=(D // SORT_CS,),
        in_specs=[pl.BlockSpec((S, SORT_CS), lambda c: (0, c))],
        out_specs=[pl.BlockSpec((CAP, SORT_CS), lambda c: (0, c)),
                   pl.BlockSpec((CAP, SORT_CS), lambda c: (0, c)),
                   pl.BlockSpec((S, SORT_CS), lambda c: (0, c))],
        out_shape=[jax.ShapeDtypeStruct((CAP, D), _f32),
                   jax.ShapeDtypeStruct((CAP, D), _i32),
                   jax.ShapeDtypeStruct((S, D), _i32)],
    )(r)


IW = 32
IC_PER_W = D // IW


def _invert_body(ifull_t, g2t, idx_v, g2_v):
    w = lax.axis_index("s") * 2 + lax.axis_index("c")
    col0 = w * IC_PER_W

    def body(t, carry):
        d = col0 + t
        pltpu.sync_copy(ifull_t.at[d], idx_v)
        for j in range(S // 16):
            c0 = j * 16
            idx_vec = idx_v[pl.ds(c0, 16)]
            cvec = lax.iota(_i32, 16) + c0
            val = jnp.where(cvec < CAP, cvec * D + d,
                            CAP * D + idx_vec * D + d)
            plsc.store_scatter(g2_v, [idx_vec], val)
        pltpu.sync_copy(g2_v, g2t.at[d])
        return carry

    lax.fori_loop(0, IC_PER_W, body, 0)


def _invert_sc(ifull_t):
    fn = pl.kernel(
        _invert_body,
        out_type=jax.ShapeDtypeStruct((D, S), _i32),
        mesh=plsc.VectorSubcoreMesh(core_axis_name="c", subcore_axis_name="s"),
        scratch_types=[pltpu.VMEM((S,), _i32),
                       pltpu.VMEM((S,), _i32)],
        compiler_params=pltpu.CompilerParams(needs_layout_passes=False),
    )
    return fn(ifull_t)


GW = 32
G_JB = 64


def _make_gather_body(n_rows):
    rows_per_w = n_rows // GW
    n_outer = rows_per_w // G_JB

    def body(src_f, gidx2, out2, idx_v, buf_v, sem):
        w = lax.axis_index("s") * 2 + lax.axis_index("c")
        row0 = w * rows_per_w

        def inner(t, carry):
            r0 = row0 + t * G_JB
            pltpu.sync_copy(gidx2.at[pl.ds(r0, G_JB)], idx_v)
            handles = [pltpu.async_copy(src_f.at[idx_v.at[j]], buf_v.at[j], sem)
                       for j in range(G_JB)]
            for h in handles:
                h.wait()
            pltpu.sync_copy(buf_v, out2.at[pl.ds(r0, G_JB)])
            return carry

        lax.fori_loop(0, n_outer, inner, 0)

    return body


def _gather_sc(src_f, gidx2):
    n_rows = gidx2.shape[0]
    fn = pl.kernel(
        _make_gather_body(n_rows),
        out_type=jax.ShapeDtypeStruct((n_rows, 128), _f32),
        mesh=plsc.VectorSubcoreMesh(core_axis_name="c", subcore_axis_name="s"),
        scratch_types=[pltpu.VMEM((G_JB, 128), _i32),
                       pltpu.VMEM((G_JB, 128), _f32),
                       pltpu.SemaphoreType.DMA],
    )
    return fn(src_f, gidx2)


def _lnmm_kernel(x_ref, g_ref, b_ref, w_ref, bias_ref, o_ref):
    x = _ln(x_ref[...], g_ref[...], b_ref[...])
    o_ref[...] = lax.dot_general(
        x, w_ref[...], (((1,), (0,)), ((), ())),
        preferred_element_type=_f32) + bias_ref[...]


def _lnmm(x, g, b, w, bias, n_out):
    BM, BN = 512, 512
    M = x.shape[0]
    return pl.pallas_call(
        _lnmm_kernel,
        grid=(M // BM, n_out // BN),
        in_specs=[pl.BlockSpec((BM, D), lambda i, j: (i, 0)),
                  pl.BlockSpec((1, D), lambda i, j: (0, 0)),
                  pl.BlockSpec((1, D), lambda i, j: (0, 0)),
                  pl.BlockSpec((D, BN), lambda i, j: (0, j)),
                  pl.BlockSpec((1, BN), lambda i, j: (0, j))],
        out_specs=pl.BlockSpec((BM, BN), lambda i, j: (i, j)),
        out_shape=jax.ShapeDtypeStruct((M, n_out), _f32),
    )(x, g, b, w, bias)


ATT_BQ = 512


def _attn_kernel(q_ref, k_ref, v_ref, o_ref):
    iq = pl.program_id(1)
    s = lax.dot_general(q_ref[...], k_ref[...], (((1,), (1,)), ((), ())),
                        preferred_element_type=_f32)
    s = s * (1.0 / float(np.sqrt(HD)))
    ri = iq * ATT_BQ + lax.broadcasted_iota(_i32, (ATT_BQ, CAP), 0)
    ci = lax.broadcasted_iota(_i32, (ATT_BQ, CAP), 1)
    s = jnp.where(ci <= ri, s, NEG_HUGE)
    m = jnp.max(s, axis=1, keepdims=True)
    e = jnp.exp(s - m)
    p = e / jnp.sum(e, axis=1, keepdims=True)
    o_ref[...] = lax.dot_general(p, v_ref[...], (((1,), (0,)), ((), ())),
                                 preferred_element_type=_f32)


def _attention(qkv):
    return pl.pallas_call(
        _attn_kernel,
        grid=(H, CAP // ATT_BQ),
        in_specs=[pl.BlockSpec((ATT_BQ, HD), lambda h, iq: (iq, h)),
                  pl.BlockSpec((CAP, HD), lambda h, iq: (0, H + h)),
                  pl.BlockSpec((CAP, HD), lambda h, iq: (0, 2 * H + h))],
        out_specs=pl.BlockSpec((ATT_BQ, HD), lambda h, iq: (iq, h)),
        out_shape=jax.ShapeDtypeStruct((CAP, D), _f32),
    )(qkv, qkv, qkv)


def _projres_kernel(x_ref, w_ref, bias_ref, res_ref, o_ref):
    o_ref[...] = lax.dot_general(
        x_ref[...], w_ref[...], (((1,), (0,)), ((), ())),
        preferred_element_type=_f32) + bias_ref[...] + res_ref[...]


def _projres(x, w, bias, res):
    BM, BN = 512, 512
    return pl.pallas_call(
        _projres_kernel,
        grid=(CAP // BM, D // BN),
        in_specs=[pl.BlockSpec((BM, D), lambda i, j: (i, 0)),
                  pl.BlockSpec((D, BN), lambda i, j: (0, j)),
                  pl.BlockSpec((1, BN), lambda i, j: (0, j)),
                  pl.BlockSpec((BM, BN), lambda i, j: (i, j))],
        out_specs=pl.BlockSpec((BM, BN), lambda i, j: (i, j)),
        out_shape=jax.ShapeDtypeStruct((CAP, D), _f32),
    )(x, w, bias, res)


def _lnfc_kernel(x_ref, g_ref, b_ref, w_ref, bias_ref, o_ref):
    x = _ln(x_ref[...], g_ref[...], b_ref[...])
    y = lax.dot_general(x, w_ref[...], (((1,), (0,)), ((), ())),
                        preferred_element_type=_f32) + bias_ref[...]
    o_ref[...] = _gelu_new(y)


def _lnfc(x, g, b, w, bias):
    BM, BN = 512, 512
    return pl.pallas_call(
        _lnfc_kernel,
        grid=(CAP // BM, INNER // BN),
        in_specs=[pl.BlockSpec((BM, D), lambda i, j: (i, 0)),
                  pl.BlockSpec((1, D), lambda i, j: (0, 0)),
                  pl.BlockSpec((1, D), lambda i, j: (0, 0)),
                  pl.BlockSpec((D, BN), lambda i, j: (0, j)),
                  pl.BlockSpec((1, BN), lambda i, j: (0, j))],
        out_specs=pl.BlockSpec((BM, BN), lambda i, j: (i, j)),
        out_shape=jax.ShapeDtypeStruct((CAP, INNER), _f32),
    )(x, g, b, w, bias)


def _mlpproj_kernel(ff_ref, w_ref, bias_ref, res_ref, vals_ref, o_ref):
    y = lax.dot_general(ff_ref[...], w_ref[...], (((1,), (0,)), ((), ())),
                        preferred_element_type=_f32) + bias_ref[...]
    o_ref[...] = res_ref[...] + y * vals_ref[...]


def _mlpproj(ff, w, bias, res, vals):
    BM, BN = 256, 256
    return pl.pallas_call(
        _mlpproj_kernel,
        grid=(CAP // BM, D // BN),
        in_specs=[pl.BlockSpec((BM, INNER), lambda i, j: (i, 0)),
                  pl.BlockSpec((INNER, BN), lambda i, j: (0, j)),
                  pl.BlockSpec((1, BN), lambda i, j: (0, j)),
                  pl.BlockSpec((BM, BN), lambda i, j: (i, j)),
                  pl.BlockSpec((BM, BN), lambda i, j: (i, j))],
        out_specs=pl.BlockSpec((BM, BN), lambda i, j: (i, j)),
        out_shape=jax.ShapeDtypeStruct((CAP, D), _f32),
    )(ff, w, bias, res, vals)


def kernel(hidden_states, W_router, ln1_g, ln1_b, c_attn_w, c_attn_b,
           attn_proj_w, attn_proj_b, ln2_g, ln2_b, c_fc_w, c_fc_b,
           mlp_proj_w, mlp_proj_b):
    hs2 = hidden_states.reshape(S, D)
    hs_f = hidden_states.reshape(S * D)

    r = _router(hs2, W_router)
    vals, gidx, ifull = _sort_topk(r)
    gidx2 = gidx.reshape(CAP * D // 128, 128)

    sel = _gather_sc(hs_f, gidx2).reshape(CAP, D)

    qkv = _lnmm(sel, ln1_g.reshape(1, D), ln1_b.reshape(1, D),
                c_attn_w, c_attn_b.reshape(1, 3 * D), 3 * D)
    ctx = _attention(qkv)
    hidden = _projres(ctx, attn_proj_w, attn_proj_b.reshape(1, D), sel)
    ff = _lnfc(hidden, ln2_g.reshape(1, D), ln2_b.reshape(1, D),
               c_fc_w, c_fc_b.reshape(1, INNER))
    hid = _mlpproj(ff, mlp_proj_w, mlp_proj_b.reshape(1, D), hidden, vals)

    g2t = _invert_sc(ifull.T)
    src_f = jnp.concatenate([hid.reshape(CAP * D), hs_f])
    out_t = _gather_sc(src_f, g2t.reshape(D * S // 128, 128))
    return out_t.reshape(D, S).T.reshape(1, S, D)


# speedup vs baseline: 3.9398x; 1.0151x over previous
import functools

import jax
import jax.numpy as jnp
import numpy as np
from jax import lax
from jax.experimental import pallas as pl
from jax.experimental.pallas import tpu as pltpu
from jax.experimental.pallas import tpu_sc as plsc

S, D, H, CAP = 4096, 2048, 16, 2048
INNER = 4 * D
HD = D // H
EPS = 1e-5
SQ2PI = 0.7978845608028654
NEG_HUGE = float(np.finfo(np.float32).min)

_f32 = jnp.float32
_i32 = jnp.int32


def _ln(x, g, b):
    mu = jnp.mean(x, axis=-1, keepdims=True)
    var = jnp.mean(jnp.square(x - mu), axis=-1, keepdims=True)
    return (x - mu) / jnp.sqrt(var + EPS) * g + b


def _gelu_new(x):
    return 0.5 * x * (1.0 + jnp.tanh(SQ2PI * (x + 0.044715 * x * x * x)))


def _router_kernel(x_ref, w_ref, o_ref):
    o_ref[...] = lax.dot_general(
        x_ref[...], w_ref[...], (((1,), (1,)), ((), ())),
        preferred_element_type=_f32)


def _router(hs2, w):
    BM, BN = 512, 512
    return pl.pallas_call(
        _router_kernel,
        grid=(S // BM, D // BN),
        in_specs=[pl.BlockSpec((BM, D), lambda i, j: (i, 0)),
                  pl.BlockSpec((BN, D), lambda i, j: (j, 0))],
        out_specs=pl.BlockSpec((BM, BN), lambda i, j: (i, j)),
        out_shape=jax.ShapeDtypeStruct((S, D), _f32),
    )(hs2, w)


SORT_CS = 128


def _sort_kernel(r_ref, vals_ref, gidx_ref, ifull_ref):
    C = r_ref.shape[1]
    col0 = pl.program_id(0) * C
    iota_s = lax.broadcasted_iota(_i32, (S, 1), 0)
    k_ = r_ref[...]
    i_ = lax.broadcasted_iota(_i32, (S, C), 0)
    kk = 2
    while kk <= S:
        jj = kk // 2
        while jj >= 1:
            up = (iota_s & jj) == 0
            xb = ((iota_s // kk) ^ (iota_s // jj)) & 1
            keep_prec = xb == 0
            pk = jnp.where(up, pltpu.roll(k_, S - jj, axis=0),
                           pltpu.roll(k_, jj, axis=0))
            pi = jnp.where(up, pltpu.roll(i_, S - jj, axis=0),
                           pltpu.roll(i_, jj, axis=0))
            prec = (k_ > pk) | ((k_ == pk) & (i_ < pi))
            swap = jnp.logical_xor(prec, keep_prec)
            k_ = jnp.where(swap, pk, k_)
            i_ = jnp.where(swap, pi, i_)
            jj //= 2
        kk *= 2
    lane = lax.broadcasted_iota(_i32, (CAP, C), 1)
    vals_ref[...] = k_[:CAP, :]
    gidx_ref[...] = i_[:CAP, :] * D + (col0 + lane)
    ifull_ref[...] = i_


def _sort_topk(r):
    return pl.pallas_call(
        _sort_kernel,
        grid=(D // SORT_CS,),
        in_specs=[pl.BlockSpec((S, SORT_CS), lambda c: (0, c))],
        out_specs=[pl.BlockSpec((CAP, SORT_CS), lambda c: (0, c)),
                   pl.BlockSpec((CAP, SORT_CS), lambda c: (0, c)),
                   pl.BlockSpec((S, SORT_CS), lambda c: (0, c))],
        out_shape=[jax.ShapeDtypeStruct((CAP, D), _f32),
                   jax.ShapeDtypeStruct((CAP, D), _i32),
                   jax.ShapeDtypeStruct((S, D), _i32)],
    )(r)


IW = 32
IC_PER_W = D // IW


def _invert_body(ifull_t, g2t, idx_v, g2_v):
    w = lax.axis_index("s") * 2 + lax.axis_index("c")
    col0 = w * IC_PER_W

    def body(t, carry):
        d = col0 + t
        pltpu.sync_copy(ifull_t.at[d], idx_v)
        for j in range(S // 16):
            c0 = j * 16
            idx_vec = idx_v[pl.ds(c0, 16)]
            cvec = lax.iota(_i32, 16) + c0
            val = jnp.where(cvec < CAP, cvec * D + d,
                            CAP * D + idx_vec * D + d)
            plsc.store_scatter(g2_v, [idx_vec], val)
        pltpu.sync_copy(g2_v, g2t.at[d])
        return carry

    lax.fori_loop(0, IC_PER_W, body, 0)


def _invert_sc(ifull_t):
    fn = pl.kernel(
        _invert_body,
        out_type=jax.ShapeDtypeStruct((D, S), _i32),
        mesh=plsc.VectorSubcoreMesh(core_axis_name="c", subcore_axis_name="s"),
        scratch_types=[pltpu.VMEM((S,), _i32),
                       pltpu.VMEM((S,), _i32)],
        compiler_params=pltpu.CompilerParams(needs_layout_passes=False),
    )
    return fn(ifull_t)


GW = 32
G_JB = 64


def _make_gather_body(n_rows):
    rows_per_w = n_rows // GW
    n_outer = rows_per_w // G_JB

    def body(src_f, gidx2, out2, idx_v, buf_v, sem):
        w = lax.axis_index("s") * 2 + lax.axis_index("c")
        row0 = w * rows_per_w

        def inner(t, carry):
            r0 = row0 + t * G_JB
            pltpu.sync_copy(gidx2.at[pl.ds(r0, G_JB)], idx_v)
            handles = [pltpu.async_copy(src_f.at[idx_v.at[j]], buf_v.at[j], sem)
                       for j in range(G_JB)]
            for h in handles:
                h.wait()
            pltpu.sync_copy(buf_v, out2.at[pl.ds(r0, G_JB)])
            return carry

        lax.fori_loop(0, n_outer, inner, 0)

    return body


def _gather_sc(src_f, gidx2):
    n_rows = gidx2.shape[0]
    fn = pl.kernel(
        _make_gather_body(n_rows),
        out_type=jax.ShapeDtypeStruct((n_rows, 128), _f32),
        mesh=plsc.VectorSubcoreMesh(core_axis_name="c", subcore_axis_name="s"),
        scratch_types=[pltpu.VMEM((G_JB, 128), _i32),
                       pltpu.VMEM((G_JB, 128), _f32),
                       pltpu.SemaphoreType.DMA],
    )
    return fn(src_f, gidx2)


def _lnmm_kernel(x_ref, g_ref, b_ref, w_ref, bias_ref, o_ref):
    x = _ln(x_ref[...], g_ref[...], b_ref[...])
    o_ref[...] = lax.dot_general(
        x.astype(jnp.bfloat16), w_ref[...], (((1,), (0,)), ((), ())),
        preferred_element_type=_f32) + bias_ref[...]


def _lnmm(x, g, b, w, bias, n_out):
    BM, BN = 512, 512
    M = x.shape[0]
    return pl.pallas_call(
        _lnmm_kernel,
        grid=(M // BM, n_out // BN),
        in_specs=[pl.BlockSpec((BM, D), lambda i, j: (i, 0)),
                  pl.BlockSpec((1, D), lambda i, j: (0, 0)),
                  pl.BlockSpec((1, D), lambda i, j: (0, 0)),
                  pl.BlockSpec((D, BN), lambda i, j: (0, j)),
                  pl.BlockSpec((1, BN), lambda i, j: (0, j))],
        out_specs=pl.BlockSpec((BM, BN), lambda i, j: (i, j)),
        out_shape=jax.ShapeDtypeStruct((M, n_out), _f32),
    )(x, g, b, w, bias)


ATT_BQ = 512


def _attn_kernel(q_ref, k_ref, v_ref, o_ref):
    iq = pl.program_id(1)
    s = lax.dot_general(q_ref[...].astype(jnp.bfloat16),
                        k_ref[...].astype(jnp.bfloat16),
                        (((1,), (1,)), ((), ())),
                        preferred_element_type=_f32)
    s = s * (1.0 / float(np.sqrt(HD)))
    ri = iq * ATT_BQ + lax.broadcasted_iota(_i32, (ATT_BQ, CAP), 0)
    ci = lax.broadcasted_iota(_i32, (ATT_BQ, CAP), 1)
    s = jnp.where(ci <= ri, s, NEG_HUGE)
    m = jnp.max(s, axis=1, keepdims=True)
    e = jnp.exp(s - m)
    p = e / jnp.sum(e, axis=1, keepdims=True)
    o_ref[...] = lax.dot_general(p.astype(jnp.bfloat16),
                                 v_ref[...].astype(jnp.bfloat16),
                                 (((1,), (0,)), ((), ())),
                                 preferred_element_type=_f32)


def _attention(qkv):
    return pl.pallas_call(
        _attn_kernel,
        grid=(H, CAP // ATT_BQ),
        in_specs=[pl.BlockSpec((ATT_BQ, HD), lambda h, iq: (iq, h)),
                  pl.BlockSpec((CAP, HD), lambda h, iq: (0, H + h)),
                  pl.BlockSpec((CAP, HD), lambda h, iq: (0, 2 * H + h))],
        out_specs=pl.BlockSpec((ATT_BQ, HD), lambda h, iq: (iq, h)),
        out_shape=jax.ShapeDtypeStruct((CAP, D), _f32),
    )(qkv, qkv, qkv)


def _projres_kernel(x_ref, w_ref, bias_ref, res_ref, o_ref):
    o_ref[...] = lax.dot_general(
        x_ref[...].astype(jnp.bfloat16), w_ref[...], (((1,), (0,)), ((), ())),
        preferred_element_type=_f32) + bias_ref[...] + res_ref[...]


def _projres(x, w, bias, res):
    BM, BN = 512, 512
    return pl.pallas_call(
        _projres_kernel,
        grid=(CAP // BM, D // BN),
        in_specs=[pl.BlockSpec((BM, D), lambda i, j: (i, 0)),
                  pl.BlockSpec((D, BN), lambda i, j: (0, j)),
                  pl.BlockSpec((1, BN), lambda i, j: (0, j)),
                  pl.BlockSpec((BM, BN), lambda i, j: (i, j))],
        out_specs=pl.BlockSpec((BM, BN), lambda i, j: (i, j)),
        out_shape=jax.ShapeDtypeStruct((CAP, D), _f32),
    )(x, w, bias, res)


def _lnfc_kernel(x_ref, g_ref, b_ref, w_ref, bias_ref, o_ref):
    x = _ln(x_ref[...], g_ref[...], b_ref[...])
    y = lax.dot_general(x.astype(jnp.bfloat16), w_ref[...],
                        (((1,), (0,)), ((), ())),
                        preferred_element_type=_f32) + bias_ref[...]
    o_ref[...] = _gelu_new(y)


def _lnfc(x, g, b, w, bias):
    BM, BN = 512, 512
    return pl.pallas_call(
        _lnfc_kernel,
        grid=(CAP // BM, INNER // BN),
        in_specs=[pl.BlockSpec((BM, D), lambda i, j: (i, 0)),
                  pl.BlockSpec((1, D), lambda i, j: (0, 0)),
                  pl.BlockSpec((1, D), lambda i, j: (0, 0)),
                  pl.BlockSpec((D, BN), lambda i, j: (0, j)),
                  pl.BlockSpec((1, BN), lambda i, j: (0, j))],
        out_specs=pl.BlockSpec((BM, BN), lambda i, j: (i, j)),
        out_shape=jax.ShapeDtypeStruct((CAP, INNER), _f32),
    )(x, g, b, w, bias)


def _mlpproj_kernel(ff_ref, w_ref, bias_ref, res_ref, vals_ref, o_ref):
    y = lax.dot_general(ff_ref[...].astype(jnp.bfloat16), w_ref[...],
                        (((1,), (0,)), ((), ())),
                        preferred_element_type=_f32) + bias_ref[...]
    o_ref[...] = res_ref[...] + y * vals_ref[...]


def _mlpproj(ff, w, bias, res, vals):
    BM, BN = 256, 256
    return pl.pallas_call(
        _mlpproj_kernel,
        grid=(CAP // BM, D // BN),
        in_specs=[pl.BlockSpec((BM, INNER), lambda i, j: (i, 0)),
                  pl.BlockSpec((INNER, BN), lambda i, j: (0, j)),
                  pl.BlockSpec((1, BN), lambda i, j: (0, j)),
                  pl.BlockSpec((BM, BN), lambda i, j: (i, j)),
                  pl.BlockSpec((BM, BN), lambda i, j: (i, j))],
        out_specs=pl.BlockSpec((BM, BN), lambda i, j: (i, j)),
        out_shape=jax.ShapeDtypeStruct((CAP, D), _f32),
    )(ff, w, bias, res, vals)


def kernel(hidden_states, W_router, ln1_g, ln1_b, c_attn_w, c_attn_b,
           attn_proj_w, attn_proj_b, ln2_g, ln2_b, c_fc_w, c_fc_b,
           mlp_proj_w, mlp_proj_b):
    hs2 = hidden_states.reshape(S, D)
    hs_f = hidden_states.reshape(S * D)

    r = _router(hs2, W_router)
    vals, gidx, ifull = _sort_topk(r)
    gidx2 = gidx.reshape(CAP * D // 128, 128)

    sel = _gather_sc(hs_f, gidx2).reshape(CAP, D)

    bf16 = jnp.bfloat16
    qkv = _lnmm(sel, ln1_g.reshape(1, D), ln1_b.reshape(1, D),
                c_attn_w.astype(bf16), c_attn_b.reshape(1, 3 * D), 3 * D)
    ctx = _attention(qkv)
    hidden = _projres(ctx, attn_proj_w.astype(bf16),
                      attn_proj_b.reshape(1, D), sel)
    ff = _lnfc(hidden, ln2_g.reshape(1, D), ln2_b.reshape(1, D),
               c_fc_w.astype(bf16), c_fc_b.reshape(1, INNER))
    hid = _mlpproj(ff, mlp_proj_w.astype(bf16),
                   mlp_proj_b.reshape(1, D), hidden, vals)

    g2t = _invert_sc(ifull.T)
    src_f = jnp.concatenate([hid.reshape(CAP * D), hs_f])
    out_t = _gather_sc(src_f, g2t.reshape(D * S // 128, 128))
    return out_t.reshape(D, S).T.reshape(1, S, D)
